# Initial kernel scaffold; baseline (speedup 1.0000x reference)
import functools

import jax
import jax.numpy as jnp
from jax import lax
from jax.experimental import pallas as pl
from jax.experimental.pallas import tpu as pltpu
from jax.experimental.pallas import tpu_sc as plsc

N_TOKENS = 16384
WORD_DIM = 128
INPUT_DIM = 2048

NC = 2
NS = 16
NW = NC * NS
B_PER_W = N_TOKENS // NW
IDX_CHUNK = 128
K_CHUNKS = B_PER_W // IDX_CHUNK


def _make_sc_gather():
    mesh = plsc.VectorSubcoreMesh(core_axis_name="c", subcore_axis_name="s")

    @functools.partial(
        pl.kernel,
        mesh=mesh,
        out_type=jax.ShapeDtypeStruct((N_TOKENS, WORD_DIM), jnp.float32),
        scratch_types=[
            pltpu.VMEM((K_CHUNKS, IDX_CHUNK), jnp.int32),
            pltpu.VMEM((B_PER_W, WORD_DIM), jnp.float32),
            pltpu.SemaphoreType.DMA,
        ],
    )
    def gather(table_hbm, idx_hbm, out_hbm, idx_v, rows_v, sem):
        wid = lax.axis_index("s") * NC + lax.axis_index("c")
        pltpu.sync_copy(idx_hbm.at[wid], idx_v)
        copies = [
            pltpu.async_copy(
                table_hbm.at[idx_v.at[j]],
                rows_v.at[pl.ds(j * IDX_CHUNK, IDX_CHUNK)],
                sem,
            )
            for j in range(K_CHUNKS)
        ]
        for c in copies:
            c.wait()
        pltpu.sync_copy(rows_v, out_hbm.at[pl.ds(wid * B_PER_W, B_PER_W)])

    return gather


_sc_gather = _make_sc_gather()


BM = 1024


def _mm_body(x_ref, w_ref, b_ref, o_ref):
    acc = jnp.dot(x_ref[...], w_ref[...], preferred_element_type=jnp.float32)
    o_ref[...] = jnp.tanh(acc + b_ref[...])


def _tc_matmul(x, w, b2d):
    grid = (N_TOKENS // BM,)
    return pl.pallas_call(
        _mm_body,
        grid=grid,
        in_specs=[
            pl.BlockSpec((BM, WORD_DIM), lambda i: (i, 0)),
            pl.BlockSpec((WORD_DIM, INPUT_DIM), lambda i: (0, 0)),
            pl.BlockSpec((1, INPUT_DIM), lambda i: (0, 0)),
        ],
        out_specs=pl.BlockSpec((BM, INPUT_DIM), lambda i: (i, 0)),
        out_shape=jax.ShapeDtypeStruct((N_TOKENS, INPUT_DIM), jnp.float32),
    )(x, w, b2d)


def kernel(word_indices, W_word, W_lin, b_lin):
    idx3 = word_indices.astype(jnp.int32).reshape(NW, K_CHUNKS, IDX_CHUNK)
    gathered = _sc_gather(W_word, idx3)
    return _tc_matmul(gathered, W_lin, b_lin.reshape(1, INPUT_DIM))


# speedup vs baseline: 1.4964x; 1.4964x over previous
import functools

import jax
import jax.numpy as jnp
from jax import lax
from jax.experimental import pallas as pl
from jax.experimental.pallas import tpu as pltpu
from jax.experimental.pallas import tpu_sc as plsc

N_TOKENS = 16384
WORD_DIM = 128
INPUT_DIM = 2048

NC = 2
NS = 16
NW = NC * NS
B_PER_W = N_TOKENS // NW
IDX_CHUNK = 128
K_CHUNKS = B_PER_W // IDX_CHUNK


@functools.lru_cache(maxsize=None)
def _make_sc_gather():
    mesh = plsc.VectorSubcoreMesh(core_axis_name="c", subcore_axis_name="s")

    @functools.partial(
        pl.kernel,
        mesh=mesh,
        out_type=jax.ShapeDtypeStruct((N_TOKENS, WORD_DIM), jnp.float32),
        scratch_types=[
            pltpu.VMEM((K_CHUNKS, IDX_CHUNK), jnp.int32),
            pltpu.VMEM((B_PER_W, WORD_DIM), jnp.float32),
            pltpu.SemaphoreType.DMA,
        ],
    )
    def gather(table_hbm, idx_hbm, out_hbm, idx_v, rows_v, sem):
        wid = lax.axis_index("s") * NC + lax.axis_index("c")
        pltpu.sync_copy(idx_hbm.at[wid], idx_v)
        copies = [
            pltpu.async_copy(
                table_hbm.at[idx_v.at[j]],
                rows_v.at[pl.ds(j * IDX_CHUNK, IDX_CHUNK)],
                sem,
            )
            for j in range(K_CHUNKS)
        ]
        for c in copies:
            c.wait()
        pltpu.sync_copy(rows_v, out_hbm.at[pl.ds(wid * B_PER_W, B_PER_W)])

    return gather


BM = 1024


def _mm_body(x_ref, w_ref, b_ref, o_ref):
    acc = jnp.dot(x_ref[...], w_ref[...], preferred_element_type=jnp.float32)
    o_ref[...] = jnp.tanh(acc + b_ref[...])


def _tc_matmul(x, w, b2d):
    grid = (N_TOKENS // BM,)
    return pl.pallas_call(
        _mm_body,
        grid=grid,
        in_specs=[
            pl.BlockSpec((BM, WORD_DIM), lambda i: (i, 0)),
            pl.BlockSpec((WORD_DIM, INPUT_DIM), lambda i: (0, 0)),
            pl.BlockSpec((1, INPUT_DIM), lambda i: (0, 0)),
        ],
        out_specs=pl.BlockSpec((BM, INPUT_DIM), lambda i: (i, 0)),
        out_shape=jax.ShapeDtypeStruct((N_TOKENS, INPUT_DIM), jnp.float32),
    )(x, w, b2d)


def kernel(word_indices, W_word, W_lin, b_lin):
    idx3 = word_indices.astype(jnp.int32).reshape(NW, K_CHUNKS, IDX_CHUNK)
    gathered = _make_sc_gather()(W_word, idx3)
    return _tc_matmul(gathered, W_lin, b_lin.reshape(1, INPUT_DIM))
